# Initial kernel scaffold; baseline (speedup 1.0000x reference)
import jax
import jax.numpy as jnp
from jax.experimental import pallas as pl


def kernel(x, W1, W2, W3, W4, W5, g1, b1, g2, b2, g3, b3, g4, b4, g5, b5, Wemb):
    raise NotImplementedError("write your pallas kernel here")


# speedup vs baseline: 11.4562x; 11.4562x over previous
import functools
import math

import jax
import jax.numpy as jnp
import numpy as np
from jax import lax
from jax.experimental import pallas as pl
from jax.experimental.pallas import tpu as pltpu

EPS = 1e-5
KNN_K = 20
SQRT1P = np.float32(np.sqrt(np.float32(1.0 + EPS)))
CW = 128


def _lrelu(y):
    return jnp.where(y >= 0.0, y, y * 0.2)


def _knn_body(xall_ref, xt_ref, idx_ref, *, n, k, tn):
    b = pl.program_id(0)
    x_all = xall_ref[0]
    xt = xt_ref[0]
    xx_all = jnp.sum(x_all * x_all, axis=1)
    xx_t = jnp.sum(xt * xt, axis=1)
    dot = lax.dot_general(xt, x_all, (((1,), (1,)), ((), ())),
                          preferred_element_type=jnp.float32)
    dist = 2.0 * dot - xx_t[:, None] - xx_all[None, :]
    iota = lax.broadcasted_iota(jnp.int32, (tn, n), 1)

    cols = []
    for t in range(k):
        m = jnp.max(dist, axis=1, keepdims=True)
        idxc = jnp.min(jnp.where(dist == m, iota, n), axis=1, keepdims=True)
        cols.append(idxc)
        if t < k - 1:
            dist = jnp.where(iota == idxc, -jnp.inf, dist)
    idx_ref[0] = jnp.concatenate(cols, axis=1) + b * n


def _knn_layer(x, *, tn=256):
    B, N, C = x.shape
    nt = N // tn
    return pl.pallas_call(
        functools.partial(_knn_body, n=N, k=KNN_K, tn=tn),
        grid=(B, nt),
        in_specs=[
            pl.BlockSpec((1, N, C), lambda b, t: (b, 0, 0)),
            pl.BlockSpec((1, tn, C), lambda b, t: (b, t, 0)),
        ],
        out_specs=pl.BlockSpec((1, tn, KNN_K), lambda b, t: (b, t, 0)),
        out_shape=jax.ShapeDtypeStruct((B, N, KNN_K), jnp.int32),
    )(x, x)


def _sc_gather(x_pad, idx_km):
    from jax.experimental.pallas import tpu_sc as plsc

    R, _ = x_pad.shape
    E = idx_km.shape[0]
    info = plsc.get_sparse_core_info()
    NW = info.num_cores * info.num_subcores
    NC = info.num_cores
    EW = E // NW
    G = 128
    NCH = EW // G
    mesh = plsc.VectorSubcoreMesh(core_axis_name="c", subcore_axis_name="s")

    @functools.partial(
        pl.kernel, mesh=mesh,
        out_type=jax.ShapeDtypeStruct((E, CW), jnp.float32),
        scratch_types=[
            pltpu.VMEM((EW,), jnp.int32),
            pltpu.VMEM((G, CW), jnp.float32),
            pltpu.VMEM((G, CW), jnp.float32),
            pltpu.SemaphoreType.DMA,
            pltpu.SemaphoreType.DMA,
        ],
    )
    def kb(x_hbm, idx_hbm, out_hbm, idx_v, buf0, buf1, sem0, sem1):
        wid = lax.axis_index("s") * NC + lax.axis_index("c")
        base = wid * EW
        pltpu.sync_copy(idx_hbm.at[pl.ds(base, EW)], idx_v)

        def start(c, buf, sem):
            pltpu.make_async_copy(
                x_hbm.at[idx_v.at[pl.ds(c * G, G)]], buf, sem).start()

        def wait(c, buf, sem):
            pltpu.make_async_copy(
                x_hbm.at[idx_v.at[pl.ds(c * G, G)]], buf, sem).wait()

        def flush(c, buf):
            pltpu.sync_copy(buf, out_hbm.at[pl.ds(base + c * G, G)])

        start(0, buf0, sem0)
        start(1, buf1, sem1)

        def step(i, carry):
            c0 = 2 * i
            wait(c0, buf0, sem0)
            flush(c0, buf0)

            @pl.when(c0 + 2 < NCH)
            def _():
                start(c0 + 2, buf0, sem0)

            wait(c0 + 1, buf1, sem1)
            flush(c0 + 1, buf1)

            @pl.when(c0 + 3 < NCH)
            def _():
                start(c0 + 3, buf1, sem1)
            return carry

        lax.fori_loop(0, NCH // 2, step, 0)

    return kb(x_pad, idx_km)


def _edge_body(xt_ref, g_ref, w_ref, gam_ref, bet_ref, out_ref, *, k, c):
    xi = xt_ref[0]
    acc = None
    for j in range(k):
        gk = g_ref[j][:, :c]
        e = jnp.concatenate([gk - xi, xi], axis=1)
        yk = lax.dot_general(e, w_ref[...], (((1,), (1,)), ((), ())),
                             preferred_element_type=jnp.float32)
        acc = yk if acc is None else jnp.maximum(acc, yk)
    y = acc / SQRT1P * gam_ref[...] + bet_ref[...]
    out_ref[0] = _lrelu(y)


def _edge_conv(x, gathered, W, gam, bet, *, tn=256):
    B, N, C = x.shape
    CO = W.shape[0]
    nt = N // tn
    return pl.pallas_call(
        functools.partial(_edge_body, k=KNN_K, c=C),
        grid=(B, nt),
        in_specs=[
            pl.BlockSpec((1, tn, C), lambda b, t: (b, t, 0)),
            pl.BlockSpec((KNN_K, tn, CW), lambda b, t, _nt=nt: (0, b * _nt + t, 0)),
            pl.BlockSpec((CO, 2 * C), lambda b, t: (0, 0)),
            pl.BlockSpec((1, CO), lambda b, t: (0, 0)),
            pl.BlockSpec((1, CO), lambda b, t: (0, 0)),
        ],
        out_specs=pl.BlockSpec((1, tn, CO), lambda b, t: (b, t, 0)),
        out_shape=jax.ShapeDtypeStruct((B, N, CO), jnp.float32),
    )(x, gathered, W, gam.reshape(1, CO), bet.reshape(1, CO))


def _edge_layer(x, W, gam, bet):
    B, N, C = x.shape
    idx = _knn_layer(x)
    idx_km = jnp.transpose(idx, (2, 0, 1)).reshape(-1)
    x_pad = jnp.pad(x.reshape(B * N, C), ((0, 0), (0, CW - C)))
    gathered = _sc_gather(x_pad, idx_km)
    return _edge_conv(x, gathered.reshape(KNN_K, B * N, CW), W, gam, bet)


def _head_body(xc_ref, w5_ref, g5_ref, b5_ref, wemb_ref, out_ref, p_scr, *, nt):
    t = pl.program_id(1)
    y = lax.dot_general(xc_ref[0], w5_ref[...], (((1,), (1,)), ((), ())),
                        preferred_element_type=jnp.float32)
    tmax = jnp.max(y, axis=0, keepdims=True)

    @pl.when(t == 0)
    def _():
        p_scr[...] = tmax

    @pl.when(t > 0)
    def _():
        p_scr[...] = jnp.maximum(p_scr[...], tmax)

    @pl.when(t == nt - 1)
    def _():
        act = _lrelu(p_scr[...] / SQRT1P * g5_ref[...] + b5_ref[...])
        out_ref[0] = lax.dot_general(act, wemb_ref[...],
                                     (((1,), (1,)), ((), ())),
                                     preferred_element_type=jnp.float32)


def _head(x_cat, W5, g5, b5, Wemb, *, tn=512):
    B, N, CAT = x_cat.shape
    F5 = W5.shape[0]
    NF = Wemb.shape[0]
    nt = N // tn
    return pl.pallas_call(
        functools.partial(_head_body, nt=nt),
        grid=(B, nt),
        in_specs=[
            pl.BlockSpec((1, tn, CAT), lambda b, t: (b, t, 0)),
            pl.BlockSpec((F5, CAT), lambda b, t: (0, 0)),
            pl.BlockSpec((1, F5), lambda b, t: (0, 0)),
            pl.BlockSpec((1, F5), lambda b, t: (0, 0)),
            pl.BlockSpec((NF, F5), lambda b, t: (0, 0)),
        ],
        out_specs=pl.BlockSpec((1, 1, NF), lambda b, t: (b, 0, 0)),
        out_shape=jax.ShapeDtypeStruct((B, 1, NF), jnp.float32),
        scratch_shapes=[pltpu.VMEM((1, F5), jnp.float32)],
    )(x_cat, W5, g5.reshape(1, F5), b5.reshape(1, F5), Wemb).reshape(B, NF)


def kernel(x, W1, W2, W3, W4, W5, g1, b1, g2, b2, g3, b3, g4, b4, g5, b5, Wemb):
    x1 = _edge_layer(x, W1, g1, b1)
    x2 = _edge_layer(x1, W2, g2, b2)
    x3 = _edge_layer(x2, W3, g3, b3)
    x4 = _edge_layer(x3, W4, g4, b4)
    x_cat = jnp.concatenate([x1, x2, x3, x4], axis=-1)
    return _head(x_cat, W5, g5, b5, Wemb)


# speedup vs baseline: 13.1280x; 1.1459x over previous
import functools
import math

import jax
import jax.numpy as jnp
import numpy as np
from jax import lax
from jax.experimental import pallas as pl
from jax.experimental.pallas import tpu as pltpu

EPS = 1e-5
KNN_K = 20
SQRT1P = np.float32(np.sqrt(np.float32(1.0 + EPS)))
CW = 128


def _lrelu(y):
    return jnp.where(y >= 0.0, y, y * 0.2)


def _knn_body(xall_ref, xt_ref, idx_ref, *, n, k, tn):
    b = pl.program_id(0)
    x_all = xall_ref[0]
    xt = xt_ref[0]
    xx_all = jnp.sum(x_all * x_all, axis=1)
    xx_t = jnp.sum(xt * xt, axis=1)
    dot = lax.dot_general(xt, x_all, (((1,), (1,)), ((), ())),
                          preferred_element_type=jnp.float32)
    dist = 2.0 * dot - xx_t[:, None] - xx_all[None, :]
    iota = lax.broadcasted_iota(jnp.int32, (tn, n), 1)

    cols = []
    for t in range(k):
        idxc = jnp.argmax(dist, axis=1).astype(jnp.int32)[:, None]
        cols.append(idxc)
        if t < k - 1:
            dist = jnp.where(iota == idxc, -jnp.inf, dist)
    idx_ref[0] = jnp.concatenate(cols, axis=1) + b * n


def _knn_layer(x, *, tn=256):
    B, N, C = x.shape
    nt = N // tn
    return pl.pallas_call(
        functools.partial(_knn_body, n=N, k=KNN_K, tn=tn),
        grid=(B, nt),
        in_specs=[
            pl.BlockSpec((1, N, C), lambda b, t: (b, 0, 0)),
            pl.BlockSpec((1, tn, C), lambda b, t: (b, t, 0)),
        ],
        out_specs=pl.BlockSpec((1, tn, KNN_K), lambda b, t: (b, t, 0)),
        out_shape=jax.ShapeDtypeStruct((B, N, KNN_K), jnp.int32),
    )(x, x)


def _sc_gather(x_pad, idx_km):
    from jax.experimental.pallas import tpu_sc as plsc

    R, _ = x_pad.shape
    E = idx_km.shape[0]
    info = plsc.get_sparse_core_info()
    NW = info.num_cores * info.num_subcores
    NC = info.num_cores
    EW = E // NW
    G = 128
    NCH = EW // G
    mesh = plsc.VectorSubcoreMesh(core_axis_name="c", subcore_axis_name="s")

    @functools.partial(
        pl.kernel, mesh=mesh,
        out_type=jax.ShapeDtypeStruct((E, CW), jnp.float32),
        scratch_types=[
            pltpu.VMEM((EW,), jnp.int32),
            pltpu.VMEM((G, CW), jnp.float32),
            pltpu.VMEM((G, CW), jnp.float32),
            pltpu.SemaphoreType.DMA,
            pltpu.SemaphoreType.DMA,
        ],
    )
    def kb(x_hbm, idx_hbm, out_hbm, idx_v, buf0, buf1, sem0, sem1):
        wid = lax.axis_index("s") * NC + lax.axis_index("c")
        base = wid * EW
        pltpu.sync_copy(idx_hbm.at[pl.ds(base, EW)], idx_v)

        def start(c, buf, sem):
            pltpu.make_async_copy(
                x_hbm.at[idx_v.at[pl.ds(c * G, G)]], buf, sem).start()

        def wait(c, buf, sem):
            pltpu.make_async_copy(
                x_hbm.at[idx_v.at[pl.ds(c * G, G)]], buf, sem).wait()

        def flush(c, buf):
            pltpu.sync_copy(buf, out_hbm.at[pl.ds(base + c * G, G)])

        start(0, buf0, sem0)
        start(1, buf1, sem1)

        def step(i, carry):
            c0 = 2 * i
            wait(c0, buf0, sem0)
            flush(c0, buf0)

            @pl.when(c0 + 2 < NCH)
            def _():
                start(c0 + 2, buf0, sem0)

            wait(c0 + 1, buf1, sem1)
            flush(c0 + 1, buf1)

            @pl.when(c0 + 3 < NCH)
            def _():
                start(c0 + 3, buf1, sem1)
            return carry

        lax.fori_loop(0, NCH // 2, step, 0)

    return kb(x_pad, idx_km)


def _edge_body(xt_ref, g_ref, w_ref, gam_ref, bet_ref, out_ref, *, k, c):
    xi = xt_ref[0]
    acc = None
    for j in range(k):
        gk = g_ref[j][:, :c]
        e = jnp.concatenate([gk - xi, xi], axis=1)
        yk = lax.dot_general(e, w_ref[...], (((1,), (1,)), ((), ())),
                             preferred_element_type=jnp.float32)
        acc = yk if acc is None else jnp.maximum(acc, yk)
    y = acc / SQRT1P * gam_ref[...] + bet_ref[...]
    out_ref[0] = _lrelu(y)


def _edge_conv(x, gathered, W, gam, bet, *, tn=256):
    B, N, C = x.shape
    CO = W.shape[0]
    nt = N // tn
    return pl.pallas_call(
        functools.partial(_edge_body, k=KNN_K, c=C),
        grid=(B, nt),
        in_specs=[
            pl.BlockSpec((1, tn, C), lambda b, t: (b, t, 0)),
            pl.BlockSpec((KNN_K, tn, CW), lambda b, t, _nt=nt: (0, b * _nt + t, 0)),
            pl.BlockSpec((CO, 2 * C), lambda b, t: (0, 0)),
            pl.BlockSpec((1, CO), lambda b, t: (0, 0)),
            pl.BlockSpec((1, CO), lambda b, t: (0, 0)),
        ],
        out_specs=pl.BlockSpec((1, tn, CO), lambda b, t: (b, t, 0)),
        out_shape=jax.ShapeDtypeStruct((B, N, CO), jnp.float32),
    )(x, gathered, W, gam.reshape(1, CO), bet.reshape(1, CO))


def _edge_layer(x, W, gam, bet):
    B, N, C = x.shape
    idx = _knn_layer(x)
    idx_km = jnp.transpose(idx, (2, 0, 1)).reshape(-1)
    x_pad = jnp.pad(x.reshape(B * N, C), ((0, 0), (0, CW - C)))
    gathered = _sc_gather(x_pad, idx_km)
    return _edge_conv(x, gathered.reshape(KNN_K, B * N, CW), W, gam, bet)


def _head_body(xc_ref, w5_ref, g5_ref, b5_ref, wemb_ref, out_ref, p_scr, *, nt):
    t = pl.program_id(1)
    y = lax.dot_general(xc_ref[0], w5_ref[...], (((1,), (1,)), ((), ())),
                        preferred_element_type=jnp.float32)
    tmax = jnp.max(y, axis=0, keepdims=True)

    @pl.when(t == 0)
    def _():
        p_scr[...] = tmax

    @pl.when(t > 0)
    def _():
        p_scr[...] = jnp.maximum(p_scr[...], tmax)

    @pl.when(t == nt - 1)
    def _():
        act = _lrelu(p_scr[...] / SQRT1P * g5_ref[...] + b5_ref[...])
        out_ref[0] = lax.dot_general(act, wemb_ref[...],
                                     (((1,), (1,)), ((), ())),
                                     preferred_element_type=jnp.float32)


def _head(x_cat, W5, g5, b5, Wemb, *, tn=512):
    B, N, CAT = x_cat.shape
    F5 = W5.shape[0]
    NF = Wemb.shape[0]
    nt = N // tn
    return pl.pallas_call(
        functools.partial(_head_body, nt=nt),
        grid=(B, nt),
        in_specs=[
            pl.BlockSpec((1, tn, CAT), lambda b, t: (b, t, 0)),
            pl.BlockSpec((F5, CAT), lambda b, t: (0, 0)),
            pl.BlockSpec((1, F5), lambda b, t: (0, 0)),
            pl.BlockSpec((1, F5), lambda b, t: (0, 0)),
            pl.BlockSpec((NF, F5), lambda b, t: (0, 0)),
        ],
        out_specs=pl.BlockSpec((1, 1, NF), lambda b, t: (b, 0, 0)),
        out_shape=jax.ShapeDtypeStruct((B, 1, NF), jnp.float32),
        scratch_shapes=[pltpu.VMEM((1, F5), jnp.float32)],
    )(x_cat, W5, g5.reshape(1, F5), b5.reshape(1, F5), Wemb).reshape(B, NF)


def kernel(x, W1, W2, W3, W4, W5, g1, b1, g2, b2, g3, b3, g4, b4, g5, b5, Wemb):
    x1 = _edge_layer(x, W1, g1, b1)
    x2 = _edge_layer(x1, W2, g2, b2)
    x3 = _edge_layer(x2, W3, g3, b3)
    x4 = _edge_layer(x3, W4, g4, b4)
    x_cat = jnp.concatenate([x1, x2, x3, x4], axis=-1)
    return _head(x_cat, W5, g5, b5, Wemb)


# speedup vs baseline: 13.1841x; 1.0043x over previous
import functools
import math

import jax
import jax.numpy as jnp
import numpy as np
from jax import lax
from jax.experimental import pallas as pl
from jax.experimental.pallas import tpu as pltpu

EPS = 1e-5
KNN_K = 20
SQRT1P = np.float32(np.sqrt(np.float32(1.0 + EPS)))
CW = 128


def _lrelu(y):
    return jnp.where(y >= 0.0, y, y * 0.2)


def _knn_body(xall_ref, xt_ref, idx_ref, *, n, k, tn):
    b = pl.program_id(0)
    x_all = xall_ref[0]
    xt = xt_ref[0]
    xx_all = jnp.sum(x_all * x_all, axis=1)
    xx_t = jnp.sum(xt * xt, axis=1)
    dot = lax.dot_general(xt, x_all, (((1,), (1,)), ((), ())),
                          preferred_element_type=jnp.float32)
    dist = 2.0 * dot - xx_t[:, None] - xx_all[None, :]
    iota = lax.broadcasted_iota(jnp.int32, (tn, n), 1)

    cols = []
    for t in range(k):
        idxc = jnp.argmax(dist, axis=1).astype(jnp.int32)[:, None]
        cols.append(idxc)
        if t < k - 1:
            dist = jnp.where(iota == idxc, -jnp.inf, dist)
    idx_ref[0] = jnp.concatenate(cols, axis=1) + b * n


def _knn_layer(x, *, tn=256):
    B, N, C = x.shape
    nt = N // tn
    return pl.pallas_call(
        functools.partial(_knn_body, n=N, k=KNN_K, tn=tn),
        grid=(B, nt),
        in_specs=[
            pl.BlockSpec((1, N, C), lambda b, t: (b, 0, 0)),
            pl.BlockSpec((1, tn, C), lambda b, t: (b, t, 0)),
        ],
        out_specs=pl.BlockSpec((1, tn, KNN_K), lambda b, t: (b, t, 0)),
        out_shape=jax.ShapeDtypeStruct((B, N, KNN_K), jnp.int32),
    )(x, x)


def _sc_gather(x_pad, idx_km):
    from jax.experimental.pallas import tpu_sc as plsc

    R, _ = x_pad.shape
    E = idx_km.shape[0]
    info = plsc.get_sparse_core_info()
    NW = info.num_cores * info.num_subcores
    NC = info.num_cores
    EW = E // NW
    G = 128
    NCH = EW // G
    mesh = plsc.VectorSubcoreMesh(core_axis_name="c", subcore_axis_name="s")

    @functools.partial(
        pl.kernel, mesh=mesh,
        out_type=jax.ShapeDtypeStruct((E, CW), jnp.float32),
        scratch_types=[
            pltpu.VMEM((EW,), jnp.int32),
            pltpu.VMEM((G, CW), jnp.float32),
            pltpu.VMEM((G, CW), jnp.float32),
            pltpu.SemaphoreType.DMA,
            pltpu.SemaphoreType.DMA,
        ],
    )
    def kb(x_hbm, idx_hbm, out_hbm, idx_v, buf0, buf1, sem0, sem1):
        wid = lax.axis_index("s") * NC + lax.axis_index("c")
        base = wid * EW
        pltpu.sync_copy(idx_hbm.at[pl.ds(base, EW)], idx_v)

        def start(c, buf, sem):
            pltpu.make_async_copy(
                x_hbm.at[idx_v.at[pl.ds(c * G, G)]], buf, sem).start()

        def wait(c, buf, sem):
            pltpu.make_async_copy(
                x_hbm.at[idx_v.at[pl.ds(c * G, G)]], buf, sem).wait()

        def flush(c, buf):
            pltpu.sync_copy(buf, out_hbm.at[pl.ds(base + c * G, G)])

        start(0, buf0, sem0)
        start(1, buf1, sem1)

        def step(i, carry):
            c0 = 2 * i
            wait(c0, buf0, sem0)
            flush(c0, buf0)

            @pl.when(c0 + 2 < NCH)
            def _():
                start(c0 + 2, buf0, sem0)

            wait(c0 + 1, buf1, sem1)
            flush(c0 + 1, buf1)

            @pl.when(c0 + 3 < NCH)
            def _():
                start(c0 + 3, buf1, sem1)
            return carry

        lax.fori_loop(0, NCH // 2, step, 0)

    return kb(x_pad, idx_km)


def _edge_body(xt_ref, g_ref, w_ref, gam_ref, bet_ref, out_ref, *, k, c):
    xi = xt_ref[0]
    acc = None
    for j in range(k):
        gk = g_ref[j][:, :c]
        e = jnp.concatenate([gk - xi, xi], axis=1)
        yk = lax.dot_general(e, w_ref[...], (((1,), (1,)), ((), ())),
                             preferred_element_type=jnp.float32)
        acc = yk if acc is None else jnp.maximum(acc, yk)
    y = acc / SQRT1P * gam_ref[...] + bet_ref[...]
    out_ref[0] = _lrelu(y)


def _edge_conv(x, gathered, W, gam, bet, *, tn=256):
    B, N, C = x.shape
    CO = W.shape[0]
    nt = N // tn
    return pl.pallas_call(
        functools.partial(_edge_body, k=KNN_K, c=C),
        grid=(B, nt),
        in_specs=[
            pl.BlockSpec((1, tn, C), lambda b, t: (b, t, 0)),
            pl.BlockSpec((KNN_K, tn, CW), lambda b, t, _nt=nt: (0, b * _nt + t, 0)),
            pl.BlockSpec((CO, 2 * C), lambda b, t: (0, 0)),
            pl.BlockSpec((1, CO), lambda b, t: (0, 0)),
            pl.BlockSpec((1, CO), lambda b, t: (0, 0)),
        ],
        out_specs=pl.BlockSpec((1, tn, CO), lambda b, t: (b, t, 0)),
        out_shape=jax.ShapeDtypeStruct((B, N, CO), jnp.float32),
    )(x, gathered, W, gam.reshape(1, CO), bet.reshape(1, CO))


def _edge_layer(x, W, gam, bet):
    B, N, C = x.shape
    idx = _knn_layer(x)
    idx_km = jnp.transpose(idx, (2, 0, 1)).reshape(-1)
    x_pad = jnp.pad(x.reshape(B * N, C), ((0, 0), (0, CW - C)))
    gathered = _sc_gather(x_pad, idx_km)
    return _edge_conv(x, gathered.reshape(KNN_K, B * N, CW), W, gam, bet)


def _head_body(xc_ref, w5_ref, g5_ref, b5_ref, wemb_ref, out_ref, p_scr, *, nt):
    t = pl.program_id(1)
    y = lax.dot_general(xc_ref[0], w5_ref[...], (((1,), (1,)), ((), ())),
                        preferred_element_type=jnp.float32)
    tmax = jnp.max(y, axis=0, keepdims=True)

    @pl.when(t == 0)
    def _():
        p_scr[...] = tmax

    @pl.when(t > 0)
    def _():
        p_scr[...] = jnp.maximum(p_scr[...], tmax)

    @pl.when(t == nt - 1)
    def _():
        act = _lrelu(p_scr[...] / SQRT1P * g5_ref[...] + b5_ref[...])
        out_ref[0] = lax.dot_general(act, wemb_ref[...],
                                     (((1,), (1,)), ((), ())),
                                     preferred_element_type=jnp.float32)


def _head(x_cat, W5, g5, b5, Wemb, *, tn=512):
    B, N, CAT = x_cat.shape
    F5 = W5.shape[0]
    NF = Wemb.shape[0]
    nt = N // tn
    return pl.pallas_call(
        functools.partial(_head_body, nt=nt),
        grid=(B, nt),
        in_specs=[
            pl.BlockSpec((1, tn, CAT), lambda b, t: (b, t, 0)),
            pl.BlockSpec((F5, CAT), lambda b, t: (0, 0)),
            pl.BlockSpec((1, F5), lambda b, t: (0, 0)),
            pl.BlockSpec((1, F5), lambda b, t: (0, 0)),
            pl.BlockSpec((NF, F5), lambda b, t: (0, 0)),
        ],
        out_specs=pl.BlockSpec((1, 1, NF), lambda b, t: (b, 0, 0)),
        out_shape=jax.ShapeDtypeStruct((B, 1, NF), jnp.float32),
        scratch_shapes=[pltpu.VMEM((1, F5), jnp.float32)],
    )(x_cat, W5, g5.reshape(1, F5), b5.reshape(1, F5), Wemb).reshape(B, NF)


def kernel(x, W1, W2, W3, W4, W5, g1, b1, g2, b2, g3, b3, g4, b4, g5, b5, Wemb):
    B = x.shape[0]
    cats = []
    for b in range(B):
        xb = x[b:b + 1]
        x1 = _edge_layer(xb, W1, g1, b1)
        x2 = _edge_layer(x1, W2, g2, b2)
        x3 = _edge_layer(x2, W3, g3, b3)
        x4 = _edge_layer(x3, W4, g4, b4)
        cats.append(jnp.concatenate([x1, x2, x3, x4], axis=-1))
    x_cat = jnp.concatenate(cats, axis=0)
    return _head(x_cat, W5, g5, b5, Wemb)
